# baseline (speedup 1.0000x reference)
import functools

import jax
import jax.numpy as jnp
from jax import lax
from jax.experimental import pallas as pl
from jax.experimental.pallas import tpu as pltpu
from jax.experimental.pallas import tpu_sc as plsc

D = 64
B = 4096
L = 16
NQ = D // L
MARGIN = 1.0

_info = plsc.get_sparse_core_info()
NC = _info.num_cores
NS = _info.num_subcores
NW = NC * NS
RPW = B // NW
CH = 8
UR = 4
NCHUNK = RPW // CH


def _lanesum(v):
    iota = lax.iota(jnp.int32, L)
    for sh in (1, 2, 4, 8):
        v = v + v.at[iota ^ sh].get(mode="promise_in_bounds")
    return v


def _rsqrt(v):
    i = lax.bitcast_convert_type(v, jnp.int32)
    i = jnp.int32(0x5F3759DF) - (i >> 1)
    y = lax.bitcast_convert_type(i, jnp.float32)
    for _ in range(3):
        y = y * (1.5 - 0.5 * v * y * y)
    return y


def _bcast_lane(vec, j):
    return jnp.broadcast_to(vec[j], (L,))


def _row_score(M, H, T, R, rr, l1f):
    z = jnp.zeros((L,), jnp.float32)
    accs = [z] * (2 * NQ)
    for g in range(D // L):
        hch = H[rr, pl.ds(g * L, L)]
        tch = T[rr, pl.ds(g * L, L)]
        for j in range(L):
            bh = _bcast_lane(hch, j)
            bt = _bcast_lane(tch, j)
            dd = g * L + j
            for q in range(NQ):
                m = M[rr, pl.ds(dd * D + q * L, L)]
                accs[q] = accs[q] + bh * m
                accs[NQ + q] = accs[NQ + q] + bt * m
    ah, at = accs[:NQ], accs[NQ:]

    def norm(a):
        ss = a[0] * a[0]
        for q in range(1, NQ):
            ss = ss + a[q] * a[q]
        sv = jnp.maximum(_lanesum(ss), jnp.float32(1e-12))
        y = _rsqrt(sv)
        return [aq * y for aq in a]

    ah = norm(ah)
    at = norm(at)
    sl1 = None
    sl2 = None
    for q in range(NQ):
        rq = R[rr, pl.ds(q * L, L)]
        dq = ah[q] + rq - at[q]
        aq = jnp.abs(dq)
        s2 = dq * dq
        sl1 = aq if sl1 is None else sl1 + aq
        sl2 = s2 if sl2 is None else sl2 + s2
    v1 = _lanesum(sl1)
    v2 = _lanesum(sl2)
    return l1f * v1 + (1.0 - l1f) * v2


def _sc_body(ph, pt, pr, nh, nt, nr, ent, rel, tr, l1h, out,
             bph, bpt, bnh, bnt, bpr, bnr,
             eph, ept, enh, ent_, erp, ern, mp, mn, l1b, accb, sem):
    c = lax.axis_index("c")
    s = lax.axis_index("s")
    wid = s * NC + c
    base = wid * RPW

    pltpu.sync_copy(ph.at[pl.ds(base, RPW)], bph)
    pltpu.sync_copy(pt.at[pl.ds(base, RPW)], bpt)
    pltpu.sync_copy(nh.at[pl.ds(base, RPW)], bnh)
    pltpu.sync_copy(nt.at[pl.ds(base, RPW)], bnt)
    pltpu.sync_copy(pr.at[pl.ds(base, RPW)], bpr)
    pltpu.sync_copy(nr.at[pl.ds(base, RPW)], bnr)
    pltpu.sync_copy(l1h, l1b)
    l1f = l1b[...]

    def chunk(ci, acc):
        c0 = ci * CH
        cps = [
            pltpu.async_copy(ent.at[bph.at[pl.ds(c0, CH)]], eph, sem),
            pltpu.async_copy(ent.at[bpt.at[pl.ds(c0, CH)]], ept, sem),
            pltpu.async_copy(ent.at[bnh.at[pl.ds(c0, CH)]], enh, sem),
            pltpu.async_copy(ent.at[bnt.at[pl.ds(c0, CH)]], ent_, sem),
            pltpu.async_copy(rel.at[bpr.at[pl.ds(c0, CH)]], erp, sem),
            pltpu.async_copy(rel.at[bnr.at[pl.ds(c0, CH)]], ern, sem),
            pltpu.async_copy(tr.at[bpr.at[pl.ds(c0, CH)]], mp, sem),
            pltpu.async_copy(tr.at[bnr.at[pl.ds(c0, CH)]], mn, sem),
        ]
        for cp in cps:
            cp.wait()

        def half(hh, acc2):
            hb = hh * UR
            for r4 in range(UR):
                r = hb + r4
                posv = _row_score(mp, eph, ept, erp, r, l1f)
                negv = _row_score(mn, enh, ent_, ern, r, l1f)
                acc2 = acc2 + jnp.maximum(posv - negv + MARGIN, 0.0)
            return acc2

        return lax.fori_loop(0, CH // UR, half, acc)

    acc = lax.fori_loop(0, NCHUNK, chunk, jnp.zeros((L,), jnp.float32))
    accb[...] = acc
    pltpu.sync_copy(accb, out.at[wid])


_sc_call = functools.partial(
    pl.kernel,
    out_type=jax.ShapeDtypeStruct((NW, L), jnp.float32),
    mesh=plsc.VectorSubcoreMesh(core_axis_name="c", subcore_axis_name="s"),
    compiler_params=pltpu.CompilerParams(use_tc_tiling_on_sc=False),
    scratch_types=(
        [pltpu.VMEM((RPW,), jnp.int32) for _ in range(6)] +
        [pltpu.VMEM((CH, D), jnp.float32) for _ in range(6)] +
        [pltpu.VMEM((CH, D * D), jnp.float32) for _ in range(2)] +
        [pltpu.VMEM((L,), jnp.float32) for _ in range(2)] +
        [pltpu.SemaphoreType.DMA]
    ),
)(_sc_body)


def kernel(x, ent_emb, rel_emb, transfer, l1_flag):
    ph = x[:, 0]
    pt = x[:, 1]
    pr = x[:, 2]
    nh = x[:, 3]
    nt = x[:, 4]
    nr = x[:, 5]
    l1v = jnp.broadcast_to(jnp.asarray(l1_flag, jnp.float32), (L,))
    out = _sc_call(ph, pt, pr, nh, nt, nr, ent_emb, rel_emb, transfer, l1v)
    return jnp.sum(out[:, 0])


# speedup vs baseline: 1.0832x; 1.0832x over previous
import functools

import jax
import jax.numpy as jnp
from jax import lax
from jax.experimental import pallas as pl
from jax.experimental.pallas import tpu as pltpu
from jax.experimental.pallas import tpu_sc as plsc

D = 64
B = 4096
REL = 1000
MARGIN = 1.0

_info = plsc.get_sparse_core_info()
NC = _info.num_cores
NS = _info.num_subcores
NW = NC * NS
RPW = B // NW

TB = 256
NT = B // TB


def _sc_gather_body(ph, pt, pr, nh, nt, nr, ent, rel, out,
                    bph, bpt, bnh, bnt, bpr, bnr,
                    g0, g1, g2, g3, g4, g5, sem):
    c = lax.axis_index("c")
    s = lax.axis_index("s")
    wid = s * NC + c
    base = wid * RPW

    pltpu.sync_copy(ph.at[pl.ds(base, RPW)], bph)
    pltpu.sync_copy(pt.at[pl.ds(base, RPW)], bpt)
    pltpu.sync_copy(nh.at[pl.ds(base, RPW)], bnh)
    pltpu.sync_copy(nt.at[pl.ds(base, RPW)], bnt)
    pltpu.sync_copy(pr.at[pl.ds(base, RPW)], bpr)
    pltpu.sync_copy(nr.at[pl.ds(base, RPW)], bnr)

    CH = 16
    cps = []
    for ci in range(RPW // CH):
        c0 = ci * CH
        sl = pl.ds(c0, CH)
        cps += [
            pltpu.async_copy(ent.at[bph.at[sl]], g0.at[sl], sem),
            pltpu.async_copy(ent.at[bpt.at[sl]], g1.at[sl], sem),
            pltpu.async_copy(ent.at[bnh.at[sl]], g2.at[sl], sem),
            pltpu.async_copy(ent.at[bnt.at[sl]], g3.at[sl], sem),
            pltpu.async_copy(rel.at[bpr.at[sl]], g4.at[sl], sem),
            pltpu.async_copy(rel.at[bnr.at[sl]], g5.at[sl], sem),
        ]
    for cp in cps:
        cp.wait()

    pltpu.sync_copy(g0, out.at[0, pl.ds(base, RPW)])
    pltpu.sync_copy(g1, out.at[1, pl.ds(base, RPW)])
    pltpu.sync_copy(g2, out.at[2, pl.ds(base, RPW)])
    pltpu.sync_copy(g3, out.at[3, pl.ds(base, RPW)])
    pltpu.sync_copy(g4, out.at[4, pl.ds(base, RPW)])
    pltpu.sync_copy(g5, out.at[5, pl.ds(base, RPW)])


_sc_gather = functools.partial(
    pl.kernel,
    out_type=jax.ShapeDtypeStruct((6, B, D), jnp.float32),
    mesh=plsc.VectorSubcoreMesh(core_axis_name="c", subcore_axis_name="s"),
    compiler_params=pltpu.CompilerParams(use_tc_tiling_on_sc=False),
    scratch_types=(
        [pltpu.VMEM((RPW,), jnp.int32) for _ in range(6)] +
        [pltpu.VMEM((RPW, D), jnp.float32) for _ in range(6)] +
        [pltpu.SemaphoreType.DMA]
    ),
)(_sc_gather_body)


def _tc_body(prel, nrel, g, tr, rm, sm, l1m, out):
    i = pl.program_id(0)
    f = l1m[0, 0]
    t_tab = tr[...]
    rmat = rm[...]
    smat = sm[...]

    def gather_mats(rcol):
        g1h = (lax.broadcasted_iota(jnp.int32, (TB, REL), 1)
               == rcol).astype(jnp.float32)
        return jnp.dot(g1h, t_tab, preferred_element_type=jnp.float32)

    def project(e, mg):
        erep = jnp.dot(e, rmat, preferred_element_type=jnp.float32)
        return jnp.dot(erep * mg, smat, preferred_element_type=jnp.float32)

    def norm(v):
        ss = jnp.maximum(jnp.sum(v * v, axis=1, keepdims=True),
                         jnp.float32(1e-12))
        return v * lax.rsqrt(ss)

    def half_score(rcol, eh, et, er):
        mg = gather_mats(rcol)
        dvec = norm(project(eh, mg)) + er - norm(project(et, mg))
        d1 = jnp.sum(jnp.abs(dvec), axis=1, keepdims=True)
        d2 = jnp.sum(dvec * dvec, axis=1, keepdims=True)
        return f * d1 + (1.0 - f) * d2

    gv = g[...]
    pos = half_score(prel[...], gv[0], gv[1], gv[4])
    neg = half_score(nrel[...], gv[2], gv[3], gv[5])
    tile_sum = jnp.sum(jnp.maximum(pos - neg + MARGIN, 0.0),
                       keepdims=True).reshape(1, 1)
    prev = jnp.where(i == 0, jnp.zeros((1, 1), jnp.float32), out[...])
    out[...] = prev + tile_sum


def _tc_call(prel, nrel, gath, transfer, rmat, smat, l1m):
    res = pl.pallas_call(
        _tc_body,
        grid=(NT,),
        in_specs=[
            pl.BlockSpec((TB, 1), lambda i: (i, 0)),
            pl.BlockSpec((TB, 1), lambda i: (i, 0)),
            pl.BlockSpec((6, TB, D), lambda i: (0, i, 0)),
            pl.BlockSpec((REL, D * D), lambda i: (0, 0)),
            pl.BlockSpec((D, D * D), lambda i: (0, 0)),
            pl.BlockSpec((D * D, D), lambda i: (0, 0)),
            pl.BlockSpec((1, 1), lambda i: (0, 0)),
        ],
        out_specs=pl.BlockSpec((1, 1), lambda i: (0, 0)),
        out_shape=jax.ShapeDtypeStruct((1, 1), jnp.float32),
    )(prel, nrel, gath, transfer, rmat, smat, l1m)
    return res[0, 0]


def kernel(x, ent_emb, rel_emb, transfer, l1_flag):
    ph = x[:, 0]
    pt = x[:, 1]
    pr = x[:, 2]
    nh = x[:, 3]
    nt = x[:, 4]
    nr = x[:, 5]

    gath = _sc_gather(ph, pt, pr, nh, nt, nr, ent_emb, rel_emb)

    cols = jnp.arange(D * D, dtype=jnp.int32)
    dd = jnp.arange(D, dtype=jnp.int32)
    rmat = (cols[None, :] // D == dd[:, None]).astype(jnp.float32)
    smat = (cols[:, None] % D == dd[None, :]).astype(jnp.float32)
    l1m = jnp.asarray(l1_flag, jnp.float32).reshape(1, 1)

    return _tc_call(pr.reshape(B, 1), nr.reshape(B, 1), gath,
                    transfer, rmat, smat, l1m)
